# baseline (speedup 1.0000x reference)
import functools

import jax
import jax.numpy as jnp
from jax import lax
from jax.experimental import pallas as pl
from jax.experimental.pallas import tpu as pltpu
from jax.experimental.pallas import tpu_sc as plsc

NUM_CORES = 2
NUM_SUBCORES = 16
LANES = 16


def kernel(input, word_table, pos_table):
    B, S = input.shape
    V, E = word_table.shape
    NW = NUM_CORES * NUM_SUBCORES
    total = B * S
    CH = 128
    n_units = total // CH
    units_w = n_units // NW
    TB = B // CH
    ET = E // 8
    idx2d = input.T.reshape(n_units, CH)
    pos_s = pos_table[:S]

    mesh = plsc.VectorSubcoreMesh(core_axis_name="c", subcore_axis_name="s")

    @functools.partial(
        pl.kernel,
        out_type=jax.ShapeDtypeStruct((S, ET, TB, 8, CH), jnp.float32),
        mesh=mesh,
        scratch_types=[
            pltpu.VMEM((units_w, CH), jnp.int32),
            pltpu.VMEM((S, E), jnp.float32),
            pltpu.VMEM((CH, E), jnp.float32),
            pltpu.VMEM((CH, E), jnp.float32),
            pltpu.VMEM((ET, 8, CH), jnp.float32),
            pltpu.VMEM((ET, 8, CH), jnp.float32),
            pltpu.SemaphoreType.DMA,
            pltpu.SemaphoreType.DMA,
            pltpu.SemaphoreType.DMA,
        ],
        compiler_params=pltpu.CompilerParams(use_tc_tiling_on_sc=False,
                                             needs_layout_passes=False),
    )
    def sc_kernel(idx_hbm, word_hbm, pos_hbm, out_hbm,
                  idx_v, pos_v, buf0_v, buf1_v, t0_v, t1_v,
                  gsem0, gsem1, ssem):
        wid = lax.axis_index("s") * NUM_CORES + lax.axis_index("c")
        ubase = wid * units_w
        bufs = (buf0_v, buf1_v)
        tbufs = (t0_v, t1_v)
        gsems = (gsem0, gsem1)
        pltpu.sync_copy(idx_hbm.at[pl.ds(ubase, units_w)], idx_v)
        pltpu.sync_copy(pos_hbm, pos_v)

        rows_j = [lax.iota(jnp.int32, LANES) + (LANES * j)
                  for j in range(CH // LANES)]

        def fire_gather(g, slot):
            pltpu.async_copy(word_hbm.at[idx_v.at[g]], bufs[slot], gsems[slot])

        def wait_gather(g, slot):
            pltpu.make_async_copy(word_hbm.at[idx_v.at[g]], bufs[slot],
                                  gsems[slot]).wait()

        def wait_store(slot):
            pltpu.make_async_copy(tbufs[slot], out_hbm.at[0, :, 0], ssem).wait()

        fire_gather(0, 0)

        def unit_pair(gp, _):
            for b in range(2):
                g = gp * 2 + b
                u = ubase + g
                srow = u // TB
                tb = u % TB

                @pl.when(g < units_w - 1)
                def _():
                    fire_gather(g + 1, 1 - b)

                @pl.when(gp >= 1)
                def _():
                    wait_store(b)

                wait_gather(g, b)

                buf, tbuf = bufs[b], tbufs[b]
                srow_vec = jnp.full((LANES,), srow, dtype=jnp.int32)

                @plsc.parallel_loop(0, ET, unroll=1)
                def _(te):
                    for e8 in range(8):
                        e = te * 8 + e8
                        col = jnp.full((LANES,), e, dtype=jnp.int32)
                        pvec = plsc.load_gather(pos_v, [srow_vec, col])
                        for j in range(CH // LANES):
                            val = plsc.load_gather(buf, [rows_j[j], col])
                            tbuf[te, e8, pl.ds(j * LANES, LANES)] = val + pvec

                pltpu.async_copy(tbuf, out_hbm.at[srow, :, tb], ssem)
            return 0

        lax.fori_loop(0, units_w // 2, unit_pair, 0)
        wait_store(0)
        wait_store(1)

    x5 = sc_kernel(idx2d, word_table, pos_s)
    return x5.transpose(2, 4, 0, 1, 3).reshape(B, S, E)


# speedup vs baseline: 1.1219x; 1.1219x over previous
import functools

import jax
import jax.numpy as jnp
from jax import lax
from jax.experimental import pallas as pl
from jax.experimental.pallas import tpu as pltpu
from jax.experimental.pallas import tpu_sc as plsc

NUM_CORES = 2
NUM_SUBCORES = 16
LANES = 16


def kernel(input, word_table, pos_table):
    B, S = input.shape
    V, E = word_table.shape
    NW = NUM_CORES * NUM_SUBCORES
    total = B * S
    CH = 128
    n_units = total // CH
    units_w = n_units // NW
    TB = B // CH
    ET = E // 8
    idx2d = input.T.reshape(n_units, CH)
    pos_s = pos_table[:S]

    mesh = plsc.VectorSubcoreMesh(core_axis_name="c", subcore_axis_name="s")

    @functools.partial(
        pl.kernel,
        out_type=jax.ShapeDtypeStruct((S, ET, TB, 8, CH), jnp.float32),
        mesh=mesh,
        scratch_types=[
            pltpu.VMEM((units_w, CH), jnp.int32),
            pltpu.VMEM((S, E), jnp.float32),
            pltpu.VMEM((CH, E), jnp.float32),
            pltpu.VMEM((CH, E), jnp.float32),
            pltpu.VMEM((ET, 8, CH), jnp.float32),
            pltpu.VMEM((ET, 8, CH), jnp.float32),
            pltpu.SemaphoreType.DMA,
            pltpu.SemaphoreType.DMA,
            pltpu.SemaphoreType.DMA,
        ],
        compiler_params=pltpu.CompilerParams(use_tc_tiling_on_sc=False,
                                             needs_layout_passes=False),
    )
    def sc_kernel(idx_hbm, word_hbm, pos_hbm, out_hbm,
                  idx_v, pos_v, buf0_v, buf1_v, t0_v, t1_v,
                  gsem0, gsem1, ssem):
        wid = lax.axis_index("s") * NUM_CORES + lax.axis_index("c")
        ubase = wid * units_w
        bufs = (buf0_v, buf1_v)
        tbufs = (t0_v, t1_v)
        gsems = (gsem0, gsem1)
        pltpu.sync_copy(idx_hbm.at[pl.ds(ubase, units_w)], idx_v)
        pltpu.sync_copy(pos_hbm, pos_v)

        lane = lax.iota(jnp.int32, LANES)
        te_c = [(lane + LANES * c) >> 3 for c in range(E // LANES)]
        e8_c = [(lane + LANES * c) & 7 for c in range(E // LANES)]

        def fire_gather(g, slot):
            pltpu.async_copy(word_hbm.at[idx_v.at[g]], bufs[slot], gsems[slot])

        def wait_gather(g, slot):
            pltpu.make_async_copy(word_hbm.at[idx_v.at[g]], bufs[slot],
                                  gsems[slot]).wait()

        def wait_store(slot):
            pltpu.make_async_copy(tbufs[slot], out_hbm.at[0, :, 0], ssem).wait()

        fire_gather(0, 0)

        def unit_pair(gp, _):
            for b in range(2):
                g = gp * 2 + b
                u = ubase + g
                srow = u // TB
                tb = u % TB

                @pl.when(g < units_w - 1)
                def _():
                    fire_gather(g + 1, 1 - b)

                @pl.when(gp >= 1)
                def _():
                    wait_store(b)

                wait_gather(g, b)

                buf, tbuf = bufs[b], tbufs[b]
                pos_c = [pos_v[srow, pl.ds(LANES * c, LANES)]
                         for c in range(E // LANES)]

                @plsc.parallel_loop(0, CH, unroll=4)
                def _(r):
                    rvec = jnp.full((LANES,), r, dtype=jnp.int32)
                    for c in range(E // LANES):
                        val = buf[r, pl.ds(LANES * c, LANES)] + pos_c[c]
                        plsc.store_scatter(tbuf, [te_c[c], e8_c[c], rvec], val)

                pltpu.async_copy(tbuf, out_hbm.at[srow, :, tb], ssem)
            return 0

        lax.fori_loop(0, units_w // 2, unit_pair, 0)
        wait_store(0)
        wait_store(1)

    x5 = sc_kernel(idx2d, word_table, pos_s)
    return x5.transpose(2, 4, 0, 1, 3).reshape(B, S, E)


# speedup vs baseline: 2.2725x; 2.0257x over previous
import functools

import jax
import jax.numpy as jnp
from jax import lax
from jax.experimental import pallas as pl
from jax.experimental.pallas import tpu as pltpu
from jax.experimental.pallas import tpu_sc as plsc

NUM_CORES = 2
NUM_SUBCORES = 16
LANES = 16


def kernel(input, word_table, pos_table):
    B, S = input.shape
    V, E = word_table.shape
    NW = NUM_CORES * NUM_SUBCORES
    total = B * S
    CH = 128
    n_units = total // CH
    units_w = n_units // NW
    TB = B // CH
    ET = E // 8
    idx2d = input.T.reshape(n_units, CH)
    pos_s = pos_table[:S]

    mesh = plsc.VectorSubcoreMesh(core_axis_name="c", subcore_axis_name="s")

    @functools.partial(
        pl.kernel,
        out_type=jax.ShapeDtypeStruct((S, ET, TB, 8, CH), jnp.float32),
        mesh=mesh,
        scratch_types=[
            pltpu.VMEM((units_w, CH), jnp.int32),
            pltpu.VMEM((S, E), jnp.float32),
            pltpu.VMEM((CH, E), jnp.float32),
            pltpu.VMEM((CH, E), jnp.float32),
            pltpu.VMEM((ET, 8, CH), jnp.float32),
            pltpu.VMEM((ET, 8, CH), jnp.float32),
            pltpu.SemaphoreType.DMA,
            pltpu.SemaphoreType.DMA,
            pltpu.SemaphoreType.DMA,
        ],
        compiler_params=pltpu.CompilerParams(use_tc_tiling_on_sc=False,
                                             needs_layout_passes=False),
    )
    def sc_kernel(idx_hbm, word_hbm, pos_hbm, out_hbm,
                  idx_v, pos_v, buf0_v, buf1_v, t0_v, t1_v,
                  gsem0, gsem1, ssem):
        wid = lax.axis_index("s") * NUM_CORES + lax.axis_index("c")
        ubase = wid * units_w
        bufs = (buf0_v, buf1_v)
        tbufs = (t0_v, t1_v)
        gsems = (gsem0, gsem1)
        pltpu.sync_copy(idx_hbm.at[pl.ds(ubase, units_w)], idx_v)
        pltpu.sync_copy(pos_hbm, pos_v)

        lane = lax.iota(jnp.int32, LANES)
        col_c = [lane + LANES * c for c in range(E // LANES)]
        te_c = [(lane + LANES * c) >> 3 for c in range(E // LANES)]
        e8_c = [(lane + LANES * c) & 7 for c in range(E // LANES)]

        def fire_gather(g, slot):
            pltpu.async_copy(word_hbm.at[idx_v.at[g]], bufs[slot], gsems[slot])

        def wait_gather(g, slot):
            pltpu.make_async_copy(word_hbm.at[idx_v.at[g]], bufs[slot],
                                  gsems[slot]).wait()

        def wait_store(slot):
            pltpu.make_async_copy(tbufs[slot], out_hbm.at[0, :, 0], ssem).wait()

        fire_gather(0, 0)

        def unit_pair(gp, _):
            for b in range(2):
                g = gp * 2 + b
                u = ubase + g
                srow = u // TB
                tb = u % TB

                @pl.when(g < units_w - 1)
                def _():
                    fire_gather(g + 1, 1 - b)

                @pl.when(gp >= 1)
                def _():
                    wait_store(b)

                wait_gather(g, b)

                buf, tbuf = bufs[b], tbufs[b]
                pos_c = [pos_v[srow, pl.ds(LANES * c, LANES)]
                         for c in range(E // LANES)]

                @plsc.parallel_loop(0, CH, unroll=4)
                def _(r):
                    rowv = (lane + r) & (CH - 1)
                    for c in range(E // LANES):
                        val = plsc.load_gather(buf, [rowv, col_c[c]]) + pos_c[c]
                        plsc.store_scatter(tbuf, [te_c[c], e8_c[c], rowv], val)

                pltpu.async_copy(tbuf, out_hbm.at[srow, :, tb], ssem)
            return 0

        lax.fori_loop(0, units_w // 2, unit_pair, 0)
        wait_store(0)
        wait_store(1)

    x5 = sc_kernel(idx2d, word_table, pos_s)
    return x5.transpose(2, 4, 0, 1, 3).reshape(B, S, E)
